# baseline (speedup 1.0000x reference)
import functools

import jax
import jax.numpy as jnp
from jax import lax
from jax.experimental import pallas as pl
from jax.experimental.pallas import tpu as pltpu
from jax.experimental.pallas import tpu_sc as plsc

_IDX_ROW = 128
_CH_ROWS = 10
_CHUNK = _IDX_ROW * _CH_ROWS
_NW = 32
_NS = 16
_L = 16
_TILE = 1024


@jax.jit
def _embed_lookup(x, table):
    b_total = x.shape[0]
    n_rows, d = table.shape
    dhi = d // 8
    n_jblk = b_total // _IDX_ROW
    b_per_w = b_total // _NW
    steps = b_per_w // _CHUNK
    assert steps % 2 == 0
    half = steps // 2
    ob_words = _CH_ROWS * 8 * _IDX_ROW

    n_full = n_rows // _IDX_ROW
    tail = n_rows - n_full * _IDX_ROW
    per_tec = -(-(n_full) // _NS)
    assert all(min(per_tec, max(n_full - s * per_tec, 0)) % 2 == 1
               for s in range(_NS))

    tview = jnp.swapaxes(table, 0, 1)

    mesh = plsc.VectorSubcoreMesh(core_axis_name="c", subcore_axis_name="s")

    @functools.partial(
        pl.kernel,
        mesh=mesh,
        out_type=(
            jax.ShapeDtypeStruct((dhi, n_jblk * _TILE), jnp.float32),
            jax.ShapeDtypeStruct((2 * n_rows, d), jnp.float32),
        ),
        scratch_types=[
            pltpu.VMEM((_CHUNK,), jnp.int32),
            pltpu.VMEM((_CHUNK,), jnp.int32),
            pltpu.VMEM((_CHUNK, d), jnp.float32),
            pltpu.VMEM((_CHUNK, d), jnp.float32),
            pltpu.VMEM((dhi, ob_words), jnp.float32),
            pltpu.VMEM((dhi, ob_words), jnp.float32),
            pltpu.VMEM((d, _IDX_ROW), jnp.float32),
            pltpu.VMEM((d, _IDX_ROW), jnp.float32),
            pltpu.VMEM((_IDX_ROW, d), jnp.float32),
            pltpu.VMEM((_IDX_ROW, d), jnp.float32),
            pltpu.SemaphoreType.DMA,
            pltpu.SemaphoreType.DMA,
            pltpu.SemaphoreType.DMA,
            pltpu.SemaphoreType.DMA,
            pltpu.SemaphoreType.DMA,
            pltpu.SemaphoreType.DMA,
        ],
        compiler_params=pltpu.CompilerParams(
            use_tc_tiling_on_sc=False, needs_layout_passes=False),
    )
    def k(x_hbm, t_hbm, out_hbm, priv_hbm,
          idx0, idx1, rows0, rows1, ob0, ob1, tb0, tb1, rb0, rb1,
          sg0, sg1, so0, so1, si0, si1):
        cid = lax.axis_index("c")
        sid = lax.axis_index("s")
        wid = sid * 2 + cid
        row0 = wid * b_per_w
        jblk0 = wid * (b_per_w // _IDX_ROW)
        cofs = cid * n_rows
        lanes = lax.iota(jnp.int32, _L)

        idx_bufs = (idx0, idx1)
        row_bufs = (rows0, rows1)
        o_bufs = (ob0, ob1)
        t_bufs = (tb0, tb1)
        r_bufs = (rb0, rb1)
        g_sems = (sg0, sg1)
        o_sems = (so0, so1)
        i_sems = (si0, si1)

        jt0 = sid * per_tec

        def t_in(jt, p):
            for h in range(d // 8):
                pltpu.async_copy(
                    t_hbm.at[pl.ds(8 * h, 8), pl.ds(jt * _IDX_ROW, _IDX_ROW)],
                    t_bufs[p].at[pl.ds(8 * h, 8)],
                    i_sems[p])

        def t_in_wait(jt, p):
            for h in range(d // 8):
                pltpu.make_async_copy(
                    t_hbm.at[pl.ds(8 * h, 8), pl.ds(jt * _IDX_ROW, _IDX_ROW)],
                    t_bufs[p].at[pl.ds(8 * h, 8)],
                    i_sems[p]).wait()

        def t_transpose(p):
            tb = t_bufs[p]
            rb = r_bufs[p]
            for b0 in range(_IDX_ROW // _L):
                bvec = lanes + b0 * _L
                for c in range(d):
                    cvec = (lanes + c) & (d - 1)
                    v = plsc.load_gather(tb, [cvec, bvec])
                    plsc.store_scatter(rb, [bvec, cvec], v)

        def t_out(jt, p):
            pltpu.async_copy(
                r_bufs[p],
                priv_hbm.at[pl.ds(cofs + jt * _IDX_ROW, _IDX_ROW)],
                o_sems[p])

        def t_out_wait(jt, p):
            pltpu.make_async_copy(
                r_bufs[p],
                priv_hbm.at[pl.ds(cofs + jt * _IDX_ROW, _IDX_ROW)],
                o_sems[p]).wait()

        @pl.when(jt0 < n_full)
        def _():
            t_in(jt0, 0)

        def p0step(jt, p, has_prev_store, fire_next):
            @pl.when(jt < n_full)
            def _():
                if fire_next:
                    @pl.when(jt + 1 < n_full)
                    def _():
                        t_in(jt + 1, 1 - p)
                t_in_wait(jt, p)
                if has_prev_store is None:
                    t_out_wait(jt, p)
                else:
                    @pl.when(has_prev_store)
                    def _():
                        t_out_wait(jt, p)
                t_transpose(p)
                t_out(jt, p)

        def p0body(g2, carry):
            jt = jt0 + g2 * 2
            p0step(jt, 0, g2 > 0, True)
            p0step(jt + 1, 1, g2 > 0, True)
            return carry

        lax.fori_loop(0, per_tec // 2, p0body, 0)
        p0step(jt0 + per_tec - 1, 0, None, False)

        t_out_wait(0, 1)
        t_out_wait(0, 0)

        if tail:
            @pl.when(sid == _NS - 1)
            def _():
                for h in range(d // 8):
                    pltpu.sync_copy(
                        t_hbm.at[pl.ds(8 * h, 8),
                                 pl.ds(n_full * _IDX_ROW, tail)],
                        t_bufs[0].at[pl.ds(8 * h, 8), pl.ds(0, tail)])
                for b0 in range(tail // _L):
                    bvec = lanes + b0 * _L
                    for c in range(d):
                        cvec = (lanes + c) & (d - 1)
                        v = plsc.load_gather(t_bufs[0], [cvec, bvec])
                        plsc.store_scatter(r_bufs[0], [bvec, cvec], v)
                pltpu.sync_copy(
                    r_bufs[0].at[pl.ds(0, tail)],
                    priv_hbm.at[pl.ds(cofs + n_full * _IDX_ROW, tail)])

        plsc.subcore_barrier()

        def fire_idx(s, p):
            pltpu.async_copy(
                x_hbm.at[pl.ds(row0 + s * _CHUNK, _CHUNK)], idx_bufs[p],
                i_sems[p])

        def wait_idx(s, p):
            pltpu.make_async_copy(
                x_hbm.at[pl.ds(row0 + s * _CHUNK, _CHUNK)], idx_bufs[p],
                i_sems[p]).wait()
            for k2 in range(_CHUNK // _L):
                sl = pl.ds(k2 * _L, _L)
                idx_bufs[p][sl] = idx_bufs[p][sl] + cofs

        def fire_gathers(p):
            for j in range(_CH_ROWS):
                pltpu.async_copy(
                    priv_hbm.at[idx_bufs[p].at[pl.ds(j * _IDX_ROW, _IDX_ROW)]],
                    row_bufs[p].at[pl.ds(j * _IDX_ROW, _IDX_ROW)],
                    g_sems[p])

        def drain_gathers(p):
            for j in range(_CH_ROWS):
                pltpu.make_async_copy(
                    priv_hbm.at[idx_bufs[p].at[pl.ds(j * _IDX_ROW, _IDX_ROW)]],
                    row_bufs[p].at[pl.ds(j * _IDX_ROW, _IDX_ROW)],
                    g_sems[p]).wait()

        def transpose_chunk(p):
            rows2 = row_bufs[p]
            ob = o_bufs[p]

            def jloop(jj, carry):
                rbase = jj * _IDX_ROW
                obase = jj * _TILE
                for b0 in range(_IDX_ROW // _L):
                    rvec = lanes + (rbase + b0 * _L)
                    for c in range(d):
                        cpl = lanes + c
                        cvec = cpl & (d - 1)
                        v = plsc.load_gather(rows2, [rvec, cvec])
                        ivec = (cpl >> 3) & 1
                        svec = ((cpl & 7) << 7) + lanes + (obase + b0 * _L)
                        plsc.store_scatter(ob, [ivec, svec], v)
                return carry

            lax.fori_loop(0, _CH_ROWS, jloop, 0)

        def fire_stores(s, p):
            w0 = (jblk0 + s * _CH_ROWS) * _TILE
            for i in range(dhi):
                pltpu.async_copy(
                    o_bufs[p].at[i],
                    out_hbm.at[i, pl.ds(w0, ob_words)],
                    o_sems[p])

        def drain_stores(s, p):
            w0 = (jblk0 + s * _CH_ROWS) * _TILE
            for i in range(dhi):
                pltpu.make_async_copy(
                    o_bufs[p].at[i],
                    out_hbm.at[i, pl.ds(w0, ob_words)],
                    o_sems[p]).wait()

        fire_idx(0, 0)
        wait_idx(0, 0)
        fire_gathers(0)
        fire_idx(1, 1)

        def body(g, carry):
            s0 = g * 2
            wait_idx(s0 + 1, 1)
            drain_gathers(0)
            fire_gathers(1)

            @pl.when(g < half - 1)
            def _():
                fire_idx(s0 + 2, 0)

            @pl.when(g > 0)
            def _():
                drain_stores(s0 - 2, 0)
            transpose_chunk(0)
            fire_stores(s0, 0)

            @pl.when(g < half - 1)
            def _():
                wait_idx(s0 + 2, 0)
            drain_gathers(1)

            @pl.when(g < half - 1)
            def _():
                fire_gathers(0)
                fire_idx(s0 + 3, 1)

            @pl.when(g > 0)
            def _():
                drain_stores(s0 - 1, 1)
            transpose_chunk(1)
            fire_stores(s0 + 1, 1)
            return carry

        lax.fori_loop(0, half, body, 0)
        drain_stores(steps - 2, 0)
        drain_stores(steps - 1, 1)

    out2d, _ = k(x, tview)
    out4d = out2d.reshape(dhi, n_jblk, 8, _IDX_ROW)
    return out4d.transpose(1, 3, 0, 2).reshape(b_total, d)


def kernel(x, table):
    return _embed_lookup(x.astype(jnp.int32), table)


# speedup vs baseline: 2.5197x; 2.5197x over previous
import functools

import jax
import jax.numpy as jnp
from jax import lax
from jax.experimental import pallas as pl
from jax.experimental.pallas import tpu as pltpu
from jax.experimental.pallas import tpu_sc as plsc

_IDX_ROW = 128
_CH_ROWS = 10
_CHUNK = _IDX_ROW * _CH_ROWS
_NW = 32
_L = 16
_TILE = 1024


@jax.jit
def _embed_lookup(x, table):
    b_total = x.shape[0]
    d = table.shape[1]
    dhi = d // 8
    n_jblk = b_total // _IDX_ROW
    b_per_w = b_total // _NW
    steps = b_per_w // _CHUNK
    assert steps % 2 == 0
    half = steps // 2
    ob_words = _CH_ROWS * 8 * _IDX_ROW

    mesh = plsc.VectorSubcoreMesh(core_axis_name="c", subcore_axis_name="s")

    @functools.partial(
        pl.kernel,
        mesh=mesh,
        out_type=jax.ShapeDtypeStruct((dhi, n_jblk * _TILE), jnp.float32),
        scratch_types=[
            pltpu.VMEM((_CHUNK,), jnp.int32),
            pltpu.VMEM((_CHUNK,), jnp.int32),
            pltpu.VMEM((_CHUNK, d), jnp.float32),
            pltpu.VMEM((_CHUNK, d), jnp.float32),
            pltpu.VMEM((_CH_ROWS * dhi * _TILE,), jnp.float32),
            pltpu.VMEM((_CH_ROWS * dhi * _TILE,), jnp.float32),
            pltpu.SemaphoreType.DMA,
            pltpu.SemaphoreType.DMA,
            pltpu.SemaphoreType.DMA,
            pltpu.SemaphoreType.DMA,
            pltpu.SemaphoreType.DMA,
            pltpu.SemaphoreType.DMA,
        ],
        compiler_params=pltpu.CompilerParams(
            use_tc_tiling_on_sc=False, needs_layout_passes=False),
    )
    def k(x_hbm, table_hbm, out_hbm, idx0, idx1, rows0, rows1, ob0, ob1,
          sg0, sg1, so0, so1, si0, si1):
        wid = lax.axis_index("s") * 2 + lax.axis_index("c")
        row0 = wid * b_per_w
        jblk0 = wid * (b_per_w // _IDX_ROW)

        idx_bufs = (idx0, idx1)
        row_bufs = (rows0, rows1)
        o_bufs = (ob0, ob1)
        g_sems = (sg0, sg1)
        o_sems = (so0, so1)
        i_sems = (si0, si1)

        def fire_idx(s, p):
            pltpu.async_copy(
                x_hbm.at[pl.ds(row0 + s * _CHUNK, _CHUNK)], idx_bufs[p],
                i_sems[p])

        def wait_idx(s, p):
            pltpu.make_async_copy(
                x_hbm.at[pl.ds(row0 + s * _CHUNK, _CHUNK)], idx_bufs[p],
                i_sems[p]).wait()

        def fire_gathers(p):
            for j in range(_CH_ROWS):
                pltpu.async_copy(
                    table_hbm.at[idx_bufs[p].at[pl.ds(j * _IDX_ROW, _IDX_ROW)]],
                    row_bufs[p].at[pl.ds(j * _IDX_ROW, _IDX_ROW)],
                    g_sems[p])

        def drain_gathers(p):
            for j in range(_CH_ROWS):
                pltpu.make_async_copy(
                    table_hbm.at[idx_bufs[p].at[pl.ds(j * _IDX_ROW, _IDX_ROW)]],
                    row_bufs[p].at[pl.ds(j * _IDX_ROW, _IDX_ROW)],
                    g_sems[p]).wait()

        def transpose_chunk(p):
            rows2 = row_bufs[p]
            ob = o_bufs[p]
            lanes = lax.iota(jnp.int32, _L)

            def jloop(jj, carry):
                rbase = jj * _IDX_ROW
                obase = jj * _TILE
                for b0 in range(_IDX_ROW // _L):
                    rvec = lanes + (rbase + b0 * _L)
                    for c in range(d):
                        cvec = (lanes + c) & (d - 1)
                        v = plsc.load_gather(rows2, [rvec, cvec])
                        svec = (cvec << 7) + lanes + (obase * dhi + b0 * _L)
                        plsc.store_scatter(ob, [svec], v)
                return carry

            lax.fori_loop(0, _CH_ROWS, jloop, 0)

        def fire_stores(s, p):
            w0 = (jblk0 + s * _CH_ROWS) * _TILE
            for i in range(dhi):
                for jj in range(_CH_ROWS):
                    pltpu.async_copy(
                        o_bufs[p].at[pl.ds((jj * dhi + i) * _TILE, _TILE)],
                        out_hbm.at[i, pl.ds(w0 + jj * _TILE, _TILE)],
                        o_sems[p])

        def drain_stores(s, p):
            w0 = (jblk0 + s * _CH_ROWS) * _TILE
            for i in range(dhi):
                for jj in range(_CH_ROWS):
                    pltpu.make_async_copy(
                        o_bufs[p].at[pl.ds((jj * dhi + i) * _TILE, _TILE)],
                        out_hbm.at[i, pl.ds(w0 + jj * _TILE, _TILE)],
                        o_sems[p]).wait()

        fire_idx(0, 0)
        wait_idx(0, 0)
        fire_gathers(0)
        fire_idx(1, 1)

        def body(g, carry):
            s0 = g * 2
            wait_idx(s0 + 1, 1)
            drain_gathers(0)
            fire_gathers(1)

            @pl.when(g < half - 1)
            def _():
                fire_idx(s0 + 2, 0)

            @pl.when(g > 0)
            def _():
                drain_stores(s0 - 2, 0)
            transpose_chunk(0)
            fire_stores(s0, 0)

            @pl.when(g < half - 1)
            def _():
                wait_idx(s0 + 2, 0)
            drain_gathers(1)

            @pl.when(g < half - 1)
            def _():
                fire_gathers(0)
                fire_idx(s0 + 3, 1)

            @pl.when(g > 0)
            def _():
                drain_stores(s0 - 1, 1)
            transpose_chunk(1)
            fire_stores(s0 + 1, 1)
            return carry

        lax.fori_loop(0, half, body, 0)
        drain_stores(steps - 2, 0)
        drain_stores(steps - 1, 1)

    out2d = k(x, table)
    out4d = out2d.reshape(dhi, n_jblk, 8, _IDX_ROW)
    return out4d.transpose(1, 3, 0, 2).reshape(b_total, d)


def kernel(x, table):
    return _embed_lookup(x.astype(jnp.int32), table)


# speedup vs baseline: 2.5421x; 1.0089x over previous
import functools

import jax
import jax.numpy as jnp
from jax import lax
from jax.experimental import pallas as pl
from jax.experimental.pallas import tpu as pltpu
from jax.experimental.pallas import tpu_sc as plsc

_IDX_ROW = 128
_CH_ROWS = 10
_CHUNK = _IDX_ROW * _CH_ROWS
_NW = 32
_L = 16
_TILE = 1024


@jax.jit
def _embed_lookup(x, table):
    b_total = x.shape[0]
    d = table.shape[1]
    dhi = d // 8
    n_jblk = b_total // _IDX_ROW
    b_per_w = b_total // _NW
    steps = b_per_w // _CHUNK
    assert steps % 2 == 0
    half = steps // 2
    ob_words = _CH_ROWS * 8 * _IDX_ROW

    mesh = plsc.VectorSubcoreMesh(core_axis_name="c", subcore_axis_name="s")

    @functools.partial(
        pl.kernel,
        mesh=mesh,
        out_type=jax.ShapeDtypeStruct((dhi, n_jblk * _TILE), jnp.float32),
        scratch_types=[
            pltpu.VMEM((_CHUNK,), jnp.int32),
            pltpu.VMEM((_CHUNK,), jnp.int32),
            pltpu.VMEM((_CHUNK, d), jnp.float32),
            pltpu.VMEM((_CHUNK, d), jnp.float32),
            pltpu.VMEM((dhi, ob_words), jnp.float32),
            pltpu.VMEM((dhi, ob_words), jnp.float32),
            pltpu.SemaphoreType.DMA,
            pltpu.SemaphoreType.DMA,
            pltpu.SemaphoreType.DMA,
            pltpu.SemaphoreType.DMA,
            pltpu.SemaphoreType.DMA,
            pltpu.SemaphoreType.DMA,
        ],
        compiler_params=pltpu.CompilerParams(
            use_tc_tiling_on_sc=False, needs_layout_passes=False),
    )
    def k(x_hbm, table_hbm, out_hbm, idx0, idx1, rows0, rows1, ob0, ob1,
          sg0, sg1, so0, so1, si0, si1):
        wid = lax.axis_index("s") * 2 + lax.axis_index("c")
        row0 = wid * b_per_w
        jblk0 = wid * (b_per_w // _IDX_ROW)

        idx_bufs = (idx0, idx1)
        row_bufs = (rows0, rows1)
        o_bufs = (ob0, ob1)
        g_sems = (sg0, sg1)
        o_sems = (so0, so1)
        i_sems = (si0, si1)

        def fire_idx(s, p):
            pltpu.async_copy(
                x_hbm.at[pl.ds(row0 + s * _CHUNK, _CHUNK)], idx_bufs[p],
                i_sems[p])

        def wait_idx(s, p):
            pltpu.make_async_copy(
                x_hbm.at[pl.ds(row0 + s * _CHUNK, _CHUNK)], idx_bufs[p],
                i_sems[p]).wait()

        def fire_gathers(p):
            for j in range(_CH_ROWS):
                pltpu.async_copy(
                    table_hbm.at[idx_bufs[p].at[pl.ds(j * _IDX_ROW, _IDX_ROW)]],
                    row_bufs[p].at[pl.ds(j * _IDX_ROW, _IDX_ROW)],
                    g_sems[p])

        def drain_gathers(p):
            for j in range(_CH_ROWS):
                pltpu.make_async_copy(
                    table_hbm.at[idx_bufs[p].at[pl.ds(j * _IDX_ROW, _IDX_ROW)]],
                    row_bufs[p].at[pl.ds(j * _IDX_ROW, _IDX_ROW)],
                    g_sems[p]).wait()

        def transpose_chunk(p):
            rows2 = row_bufs[p]
            ob = o_bufs[p]
            lanes = lax.iota(jnp.int32, _L)

            def jloop(jj, carry):
                rbase = jj * _IDX_ROW
                obase = jj * _TILE
                for b0 in range(_IDX_ROW // _L):
                    rvec = lanes + (rbase + b0 * _L)
                    for c in range(d):
                        cpl = lanes + c
                        cvec = cpl & (d - 1)
                        v = plsc.load_gather(rows2, [rvec, cvec])
                        ivec = (cpl >> 3) & 1
                        svec = ((cpl & 7) << 7) + lanes + (obase + b0 * _L)
                        plsc.store_scatter(ob, [ivec, svec], v)
                return carry

            lax.fori_loop(0, _CH_ROWS, jloop, 0)

        def fire_stores(s, p):
            w0 = (jblk0 + s * _CH_ROWS) * _TILE
            for i in range(dhi):
                pltpu.async_copy(
                    o_bufs[p].at[i],
                    out_hbm.at[i, pl.ds(w0, ob_words)],
                    o_sems[p])

        def drain_stores(s, p):
            w0 = (jblk0 + s * _CH_ROWS) * _TILE
            for i in range(dhi):
                pltpu.make_async_copy(
                    o_bufs[p].at[i],
                    out_hbm.at[i, pl.ds(w0, ob_words)],
                    o_sems[p]).wait()

        fire_idx(0, 0)
        wait_idx(0, 0)
        fire_gathers(0)
        fire_idx(1, 1)

        def body(g, carry):
            s0 = g * 2
            wait_idx(s0 + 1, 1)
            drain_gathers(0)
            fire_gathers(1)

            @pl.when(g < half - 1)
            def _():
                fire_idx(s0 + 2, 0)

            @pl.when(g > 0)
            def _():
                drain_stores(s0 - 2, 0)
            transpose_chunk(0)
            fire_stores(s0, 0)

            @pl.when(g < half - 1)
            def _():
                wait_idx(s0 + 2, 0)
            drain_gathers(1)

            @pl.when(g < half - 1)
            def _():
                fire_gathers(0)
                fire_idx(s0 + 3, 1)

            @pl.when(g > 0)
            def _():
                drain_stores(s0 - 1, 1)
            transpose_chunk(1)
            fire_stores(s0 + 1, 1)
            return carry

        lax.fori_loop(0, half, body, 0)
        drain_stores(steps - 2, 0)
        drain_stores(steps - 1, 1)

    out2d = k(x, table)
    out4d = out2d.reshape(dhi, n_jblk, 8, _IDX_ROW)
    return out4d.transpose(1, 3, 0, 2).reshape(b_total, d)


def kernel(x, table):
    return _embed_lookup(x.astype(jnp.int32), table)
